# baseline (speedup 1.0000x reference)
import functools

import jax
import jax.numpy as jnp
from jax import lax
from jax.experimental import pallas as pl
from jax.experimental.pallas import tpu as pltpu
from jax.experimental.pallas import tpu_sc as plsc

B = 1024
COUNT_DIM = 8
NUM_PROPS = 1032
FP_DIM = NUM_PROPS - COUNT_DIM
N_EMBD = 64
K = NUM_PROPS // 2
NC, NS = 2, 16
NW = NC * NS
BPW = B // NW
PAIRS = BPW * K
S = 760
NFULL = PAIRS // S
TAIL = PAIRS - NFULL * S
TAIL_OFF = NFULL * S

_mesh = plsc.VectorSubcoreMesh(core_axis_name="c", subcore_axis_name="s")


@functools.partial(
    pl.kernel,
    mesh=_mesh,
    out_type=jax.ShapeDtypeStruct((B * K, 2 * N_EMBD), jnp.float32),
    scratch_types=[
        pltpu.VMEM((PAIRS,), jnp.int32),
        pltpu.VMEM((S, 2 * N_EMBD), jnp.float32),
        pltpu.VMEM_SHARED((4 * K, 2 * N_EMBD), jnp.float32),
        pltpu.SemaphoreType.DMA,
    ],
)
def _sc_embed(q_hbm, table_hbm, out_hbm, idx_v, buf_a, table_s, sem_g):
    sid = lax.axis_index("s")
    wid = lax.axis_index("c") * NS + sid
    slot0 = wid * PAIRS

    @pl.when(sid == 0)
    def _():
        pltpu.sync_copy(table_hbm, table_s)

    pltpu.sync_copy(q_hbm.at[pl.ds(slot0, PAIRS)], idx_v)

    def per_chunk(c, k):
        idx_v[pl.ds(c * 16, 16)] = idx_v[pl.ds(c * 16, 16)] * K + k
        k = k + 16
        return jnp.where(k >= K, k - K, k)

    lax.fori_loop(0, PAIRS // 16, per_chunk,
                  lax.broadcasted_iota(jnp.int32, (16,), 0))
    plsc.subcore_barrier()

    def chunks_of(total):
        out, off = [], 0
        while off < total:
            n = min(128, total - off)
            out.append((off, n))
            off += n
        return tuple(out)

    def fire_gathers(base, buf_v, total):
        return [
            pltpu.async_copy(
                table_s.at[idx_v.at[pl.ds(base + off, n)]],
                buf_v.at[pl.ds(off, n)],
                sem_g,
            )
            for off, n in chunks_of(total)
        ]

    def run_stage(base, total):
        for cp in fire_gathers(base, buf_a, total):
            cp.wait()
        pltpu.sync_copy(buf_a.at[pl.ds(0, total)],
                        out_hbm.at[pl.ds(slot0 + base, total)])

    def per_stage(j, carry):
        run_stage(j * S, S)
        return carry

    lax.fori_loop(0, NFULL, per_stage, 0)
    run_stage(TAIL_OFF, TAIL)


def _build_table(type_emb, count_val, count_bit, fp_pair, fp_bit, fp_val):
    base_c = count_val[0] + count_bit + type_emb[0]
    base_f = (fp_val[0]
              + jnp.repeat(fp_pair, 2, axis=0)
              + jnp.tile(fp_bit, (FP_DIM // 2, 1))
              + type_emb[1])
    base = jnp.concatenate([base_c, base_f], axis=0)
    delta_c = jnp.broadcast_to(count_val[1] - count_val[0],
                               (COUNT_DIM, N_EMBD))
    delta_f = jnp.broadcast_to(fp_val[1] - fp_val[0], (FP_DIM, N_EMBD))
    delta = jnp.concatenate([delta_c, delta_f], axis=0)
    full = jnp.stack([base, base + delta])
    even = full[:, 0::2, :]
    odd = full[:, 1::2, :]
    paired = jnp.concatenate([
        jnp.broadcast_to(even[:, None], (2, 2, K, N_EMBD)),
        jnp.broadcast_to(odd[None, :], (2, 2, K, N_EMBD)),
    ], axis=-1)
    return paired.reshape(4 * K, 2 * N_EMBD)


def kernel(prop, type_emb, count_val, count_bit, fp_pair, fp_bit, fp_val):
    table = _build_table(type_emb, count_val, count_bit, fp_pair, fp_bit,
                         fp_val)
    q = 2 * prop[:, 0::2] + prop[:, 1::2]
    out = _sc_embed(q.reshape(-1), table)
    return out.reshape(B, NUM_PROPS, N_EMBD)


# speedup vs baseline: 1.8654x; 1.8654x over previous
import functools

import jax
import jax.numpy as jnp
from jax import lax
from jax.experimental import pallas as pl
from jax.experimental.pallas import tpu as pltpu
from jax.experimental.pallas import tpu_sc as plsc

B = 1024
COUNT_DIM = 8
NUM_PROPS = 1032
FP_DIM = NUM_PROPS - COUNT_DIM
N_EMBD = 64
K = NUM_PROPS // 2
KP = 528
NC, NS = 2, 16
NW = NC * NS
BPW = B // NW

GATHER_CHUNKS = ((0, 128), (128, 128), (256, 128), (384, 128), (512, 4))

_mesh = plsc.VectorSubcoreMesh(core_axis_name="c", subcore_axis_name="s")


@functools.partial(
    pl.kernel,
    mesh=_mesh,
    out_type=jax.ShapeDtypeStruct((B, K, 2 * N_EMBD), jnp.float32),
    scratch_types=[
        pltpu.VMEM((BPW * KP,), jnp.int32),
        pltpu.VMEM((KP,), jnp.int32),
        pltpu.VMEM((KP,), jnp.int32),
        pltpu.VMEM((K, 2 * N_EMBD), jnp.float32),
        pltpu.VMEM_SHARED((4 * K, 2 * N_EMBD), jnp.float32),
        pltpu.SemaphoreType.DMA,
    ],
)
def _sc_embed(q_hbm, table_hbm, out_hbm, q_v, idx_a, idx_b, row_v, table_s,
              sem_g):
    sid = lax.axis_index("s")
    wid = lax.axis_index("c") * NS + sid
    row0 = wid * BPW

    @pl.when(sid == 0)
    def _():
        pltpu.sync_copy(table_hbm, table_s)

    pltpu.sync_copy(q_hbm.at[pl.ds(row0 * KP, BPW * KP)], q_v)
    plsc.subcore_barrier()

    def build_idx(idx_v, i):
        i = jnp.minimum(i, BPW - 1)

        def per_chunk(c, k):
            idx_v[pl.ds(c * 16, 16)] = q_v[pl.ds(i * KP + c * 16, 16)] * K + k
            return k + 16

        lax.fori_loop(0, KP // 16, per_chunk,
                      lax.broadcasted_iota(jnp.int32, (16,), 0))

    def do_row(i, idx_cur, idx_nxt):
        gathers = [
            pltpu.async_copy(
                table_s.at[idx_cur.at[pl.ds(off, n)]],
                row_v.at[pl.ds(off, n)],
                sem_g,
            )
            for off, n in GATHER_CHUNKS
        ]
        build_idx(idx_nxt, i + 1)
        for cp in gathers:
            cp.wait()
        pltpu.sync_copy(row_v, out_hbm.at[row0 + i])

    build_idx(idx_a, 0)

    def per_pair(j, carry):
        do_row(2 * j, idx_a, idx_b)
        do_row(2 * j + 1, idx_b, idx_a)
        return carry

    lax.fori_loop(0, BPW // 2, per_pair, 0)


def _build_table(type_emb, count_val, count_bit, fp_pair, fp_bit, fp_val):
    base_c = count_val[0] + count_bit + type_emb[0]
    base_f = (fp_val[0]
              + jnp.repeat(fp_pair, 2, axis=0)
              + jnp.tile(fp_bit, (FP_DIM // 2, 1))
              + type_emb[1])
    base = jnp.concatenate([base_c, base_f], axis=0)
    delta_c = jnp.broadcast_to(count_val[1] - count_val[0],
                               (COUNT_DIM, N_EMBD))
    delta_f = jnp.broadcast_to(fp_val[1] - fp_val[0], (FP_DIM, N_EMBD))
    delta = jnp.concatenate([delta_c, delta_f], axis=0)
    full = jnp.stack([base, base + delta])
    even = full[:, 0::2, :]
    odd = full[:, 1::2, :]
    paired = jnp.concatenate([
        jnp.broadcast_to(even[:, None], (2, 2, K, N_EMBD)),
        jnp.broadcast_to(odd[None, :], (2, 2, K, N_EMBD)),
    ], axis=-1)
    return paired.reshape(4 * K, 2 * N_EMBD)


def kernel(prop, type_emb, count_val, count_bit, fp_pair, fp_bit, fp_val):
    table = _build_table(type_emb, count_val, count_bit, fp_pair, fp_bit,
                         fp_val)
    q = 2 * prop[:, 0::2] + prop[:, 1::2]
    q = jnp.concatenate(
        [q, jnp.zeros((B, KP - K), jnp.int32)], axis=1)
    out = _sc_embed(q.reshape(-1), table)
    return out.reshape(B, NUM_PROPS, N_EMBD)


# speedup vs baseline: 2.0504x; 1.0992x over previous
import functools

import jax
import jax.numpy as jnp
from jax import lax
from jax.experimental import pallas as pl
from jax.experimental.pallas import tpu as pltpu
from jax.experimental.pallas import tpu_sc as plsc

B = 1024
COUNT_DIM = 8
NUM_PROPS = 1032
FP_DIM = NUM_PROPS - COUNT_DIM
N_EMBD = 64
K = NUM_PROPS // 2
H = K // 2
HP = 272
NC, NS = 2, 16
NW = NC * NS
BPW = B // NW

GATHER_CHUNKS = ((0, 128), (128, 128), (256, 2))

_mesh = plsc.VectorSubcoreMesh(core_axis_name="c", subcore_axis_name="s")


@functools.partial(
    pl.kernel,
    mesh=_mesh,
    out_type=jax.ShapeDtypeStruct((B, 2, H, 2 * N_EMBD), jnp.float32),
    scratch_types=[
        pltpu.VMEM((BPW * 2 * HP,), jnp.int32),
        pltpu.VMEM((HP,), jnp.int32),
        pltpu.VMEM((HP,), jnp.int32),
        pltpu.VMEM((HP, 2 * N_EMBD), jnp.float32),
        pltpu.VMEM((HP, 2 * N_EMBD), jnp.float32),
        pltpu.VMEM_SHARED((4 * K, 2 * N_EMBD), jnp.float32),
        pltpu.SemaphoreType.DMA,
        pltpu.SemaphoreType.DMA,
        pltpu.SemaphoreType.DMA,
    ],
)
def _sc_embed(q_hbm, table_hbm, out_hbm, q_v, idx_a, idx_b, buf_a, buf_b,
              table_s, sem_g, sem_a, sem_b):
    sid = lax.axis_index("s")
    wid = lax.axis_index("c") * NS + sid
    row0 = wid * BPW

    @pl.when(sid == 0)
    def _():
        pltpu.sync_copy(table_hbm, table_s)

    pltpu.sync_copy(q_hbm.at[pl.ds(row0 * 2 * HP, BPW * 2 * HP)], q_v)
    plsc.subcore_barrier()

    def build_idx(idx_v, t):
        t = jnp.minimum(t, 2 * BPW - 1)
        k0 = (t % 2) * H

        def per_chunk(c, k):
            idx_v[pl.ds(c * 16, 16)] = q_v[pl.ds(t * HP + c * 16, 16)] * K + k
            return k + 16

        lax.fori_loop(0, HP // 16, per_chunk,
                      k0 + lax.broadcasted_iota(jnp.int32, (16,), 0))

    build_idx(idx_a, 0)

    def per_row(i, carry):
        @pl.when(i > 0)
        def _():
            pltpu.make_async_copy(out_hbm.at[0, 0], buf_a.at[pl.ds(0, H)],
                                  sem_a).wait()
        gathers = [
            pltpu.async_copy(
                table_s.at[idx_a.at[pl.ds(off, n)]],
                buf_a.at[pl.ds(off, n)],
                sem_g,
            )
            for off, n in GATHER_CHUNKS
        ]
        build_idx(idx_b, 2 * i + 1)
        for cp in gathers:
            cp.wait()
        pltpu.async_copy(buf_a.at[pl.ds(0, H)], out_hbm.at[row0 + i, 0],
                         sem_a)

        @pl.when(i > 0)
        def _():
            pltpu.make_async_copy(out_hbm.at[0, 0], buf_b.at[pl.ds(0, H)],
                                  sem_b).wait()
        gathers = [
            pltpu.async_copy(
                table_s.at[idx_b.at[pl.ds(off, n)]],
                buf_b.at[pl.ds(off, n)],
                sem_g,
            )
            for off, n in GATHER_CHUNKS
        ]
        build_idx(idx_a, 2 * i + 2)
        for cp in gathers:
            cp.wait()
        pltpu.async_copy(buf_b.at[pl.ds(0, H)], out_hbm.at[row0 + i, 1],
                         sem_b)
        return carry

    lax.fori_loop(0, BPW, per_row, 0)
    pltpu.make_async_copy(out_hbm.at[0, 0], buf_a.at[pl.ds(0, H)],
                          sem_a).wait()
    pltpu.make_async_copy(out_hbm.at[0, 0], buf_b.at[pl.ds(0, H)],
                          sem_b).wait()


def _build_table(type_emb, count_val, count_bit, fp_pair, fp_bit, fp_val):
    base_c = count_val[0] + count_bit + type_emb[0]
    base_f = (fp_val[0]
              + jnp.repeat(fp_pair, 2, axis=0)
              + jnp.tile(fp_bit, (FP_DIM // 2, 1))
              + type_emb[1])
    base = jnp.concatenate([base_c, base_f], axis=0)
    delta_c = jnp.broadcast_to(count_val[1] - count_val[0],
                               (COUNT_DIM, N_EMBD))
    delta_f = jnp.broadcast_to(fp_val[1] - fp_val[0], (FP_DIM, N_EMBD))
    delta = jnp.concatenate([delta_c, delta_f], axis=0)
    full = jnp.stack([base, base + delta])
    even = full[:, 0::2, :]
    odd = full[:, 1::2, :]
    paired = jnp.concatenate([
        jnp.broadcast_to(even[:, None], (2, 2, K, N_EMBD)),
        jnp.broadcast_to(odd[None, :], (2, 2, K, N_EMBD)),
    ], axis=-1)
    return paired.reshape(4 * K, 2 * N_EMBD)


def kernel(prop, type_emb, count_val, count_bit, fp_pair, fp_bit, fp_val):
    table = _build_table(type_emb, count_val, count_bit, fp_pair, fp_bit,
                         fp_val)
    q = 2 * prop[:, 0::2] + prop[:, 1::2]
    q = jnp.pad(q.reshape(B, 2, H), ((0, 0), (0, 0), (0, HP - H)))
    out = _sc_embed(q.reshape(-1), table)
    return out.reshape(B, NUM_PROPS, N_EMBD)
